# baseline (speedup 1.0000x reference)
import functools

import jax
import jax.numpy as jnp
from jax.experimental import pallas as pl
from jax.experimental.pallas import tpu as pltpu


def _se_fused_kernel(x_ref, w1t_ref, b1_ref, w2t_ref, b2_ref, o_ref,
                     *, nb, hw, inv_hw):
    x = x_ref[...]
    c = x.shape[1]
    x3 = x.reshape(nb, hw, c)
    pooled = jnp.sum(x3, axis=1, dtype=jnp.float32) * inv_hw
    h = jnp.dot(pooled, w1t_ref[...],
                preferred_element_type=jnp.float32) + b1_ref[...]
    h = jnp.maximum(h, 0.0)
    s = jnp.dot(h, w2t_ref[...],
                preferred_element_type=jnp.float32) + b2_ref[...]
    s = jax.nn.sigmoid(s)
    o_ref[...] = (x3 * s[:, None, :]).reshape(nb * hw, c).astype(o_ref.dtype)


def kernel(x_nchw, w1, b1, w2, b2):
    N, C, H, W = x_nchw.shape
    R = w1.shape[0]
    HW = H * W
    NB = 4 if N % 4 == 0 else (2 if N % 2 == 0 else 1)

    x2 = jnp.transpose(x_nchw, (0, 2, 3, 1)).reshape(N * HW, C)
    w1t = jnp.transpose(w1)
    w2t = jnp.transpose(w2)
    b1r = b1.reshape(1, R).astype(jnp.float32)
    b2r = b2.reshape(1, C).astype(jnp.float32)

    out = pl.pallas_call(
        functools.partial(_se_fused_kernel, nb=NB, hw=HW, inv_hw=1.0 / HW),
        out_shape=jax.ShapeDtypeStruct((N * HW, C), x_nchw.dtype),
        grid=(N // NB,),
        in_specs=[
            pl.BlockSpec((NB * HW, C), lambda n: (n, 0)),
            pl.BlockSpec((C, R), lambda n: (0, 0)),
            pl.BlockSpec((1, R), lambda n: (0, 0)),
            pl.BlockSpec((R, C), lambda n: (0, 0)),
            pl.BlockSpec((1, C), lambda n: (0, 0)),
        ],
        out_specs=pl.BlockSpec((NB * HW, C), lambda n: (n, 0)),
        compiler_params=pltpu.CompilerParams(
            dimension_semantics=("parallel",),
            vmem_limit_bytes=64 * 1024 * 1024),
    )(x2, w1t, b1r, w2t, b2r)

    return jnp.transpose(out.reshape(N, H, W, C), (0, 3, 1, 2))
